# baseline (speedup 1.0000x reference)
import functools

import jax
import jax.numpy as jnp
from jax import lax
from jax.experimental import pallas as pl
from jax.experimental.pallas import tpu as pltpu
from jax.experimental.pallas import tpu_sc as plsc

_VOCAB = 5000
_D = 16
_BATCH = 16384
_HIST = 200
_B = _BATCH * _HIST
_NW = 32
_RPS = 128
_CHUNK = 512
_SUB = _CHUNK // _RPS
_PER_W = _B // _NW
_STEPS = _PER_W // _CHUNK
_IDX_ROWS_PER_W = _PER_W // _RPS


def _sc_gather_body(table_hbm, idx_hbm, out_hbm, idx_v, rows_v, gsem, osem):
    wid = lax.axis_index("s") * 2 + lax.axis_index("c")
    out_base = wid * _PER_W

    pltpu.sync_copy(idx_hbm.at[pl.ds(wid * _IDX_ROWS_PER_W, _IDX_ROWS_PER_W)],
                    idx_v)

    def out_copy(i, b):
        return pltpu.make_async_copy(
            rows_v.at[b],
            out_hbm.at[pl.ds(out_base + i * _CHUNK, _CHUNK)],
            osem)

    @pl.loop(0, _STEPS, step=2)
    def steps(g):
        for b in range(2):
            i = g + b
            @pl.when(i >= 2)
            def _():
                out_copy(i - 2, b).wait()
            copies = [
                pltpu.async_copy(
                    table_hbm.at[idx_v.at[i * _SUB + j]],
                    rows_v.at[b, pl.ds(j * _RPS, _RPS)],
                    gsem)
                for j in range(_SUB)
            ]
            for c in copies:
                c.wait()
            out_copy(i, b).start()

    out_copy(_STEPS - 2, 0).wait()
    out_copy(_STEPS - 1, 1).wait()


@functools.cache
def _sc_gather():
    return pl.kernel(
        _sc_gather_body,
        out_type=jax.ShapeDtypeStruct((_B, _D), jnp.float32),
        mesh=plsc.VectorSubcoreMesh(core_axis_name="c", subcore_axis_name="s"),
        scratch_types=[
            pltpu.VMEM((_IDX_ROWS_PER_W, _RPS), jnp.int32),
            pltpu.VMEM((2, _CHUNK, _D), jnp.float32),
            pltpu.SemaphoreType.DMA,
            pltpu.SemaphoreType.DMA,
        ],
        compiler_params=pltpu.CompilerParams(use_tc_tiling_on_sc=False),
    )


def kernel(batch_data, table):
    idx = batch_data.astype(jnp.int32).reshape(_B // _RPS, _RPS)
    out = _sc_gather()(table, idx)
    return out.reshape(_BATCH, _HIST, _D)
